# Initial kernel scaffold; baseline (speedup 1.0000x reference)
import jax
import jax.numpy as jnp
from jax.experimental import pallas as pl


def kernel(x, edge_index, W1, b1, W2, b2, W3, b3, Wd1, bd1, Wd2, bd2, Wd3, bd3, Ws1, bs1, Ws2, bs2, Wf, bf):
    raise NotImplementedError("write your pallas kernel here")


# speedup vs baseline: 6.4825x; 6.4825x over previous
import functools

import jax
import jax.numpy as jnp
from jax import lax
from jax.experimental import pallas as pl
from jax.experimental.pallas import tpu as pltpu
from jax.experimental.pallas import tpu_sc as plsc

N = 10000
NP = 10240
E = 320000
NC = 2
NS = 16
ROWS_PT = NP // NS
KB = 80
BM = 256

_mesh = plsc.VectorSubcoreMesh(core_axis_name="c", subcore_axis_name="s")


def _build_deg():
    TPC = E // (NC * NS)
    NB = TPC // KB

    @functools.partial(
        pl.kernel,
        out_type=jax.ShapeDtypeStruct((2, NP), jnp.float32),
        mesh=_mesh,
        scratch_types=[
            pltpu.VMEM((KB,), jnp.int32),
            pltpu.VMEM((KB,), jnp.float32),
            pltpu.VMEM((ROWS_PT,), jnp.float32),
            pltpu.VMEM_SHARED((NP,), jnp.float32),
            pltpu.SemaphoreType.DMA,
        ],
    )
    def deg_k(dst_hbm, deg_hbm, dst_v, ones_v, loc_v, acc, sem):
        core = lax.axis_index("c")
        sid = lax.axis_index("s")
        w = core * NS + sid
        rbase = sid * ROWS_PT

        for i in range(KB // 16):
            ones_v[pl.ds(i * 16, 16)] = jnp.ones((16,), jnp.float32)

        def zero_body(i, carry):
            loc_v[pl.ds(i * 16, 16)] = jnp.zeros((16,), jnp.float32)
            return carry

        lax.fori_loop(0, ROWS_PT // 16, zero_body, 0)
        pltpu.sync_copy(loc_v, acc.at[pl.ds(rbase, ROWS_PT)])
        plsc.subcore_barrier()

        def edge_body(j, carry):
            eb = w * TPC + j * KB
            pltpu.sync_copy(dst_hbm.at[pl.ds(eb, KB)], dst_v)
            pltpu.sync_copy(ones_v, acc.at[dst_v], add=True)
            return carry

        lax.fori_loop(0, NB, edge_body, 0)
        plsc.subcore_barrier()
        pltpu.sync_copy(acc.at[pl.ds(rbase, ROWS_PT)],
                        deg_hbm.at[core, pl.ds(rbase, ROWS_PT)])

    return deg_k


_deg = _build_deg()


def _dinv_tc(deg3):
    def body(d_ref, o_ref):
        s = d_ref[0] + d_ref[1] + 1.0
        o_ref[...] = lax.rsqrt(s)

    return pl.pallas_call(
        body,
        grid=(NP // BM,),
        in_specs=[pl.BlockSpec((2, BM, 1), lambda m: (0, m, 0))],
        out_specs=pl.BlockSpec((BM, 1), lambda m: (m, 0)),
        out_shape=jax.ShapeDtypeStruct((NP, 1), jnp.float32),
    )(deg3)


def _build_prop(C):
    TPC = E // (NC * NS)
    NB = TPC // KB

    @functools.partial(
        pl.kernel,
        out_type=jax.ShapeDtypeStruct((C, 2, NP, 128), jnp.float32),
        mesh=_mesh,
        scratch_types=[
            pltpu.VMEM((KB,), jnp.int32),
            pltpu.VMEM((KB,), jnp.int32),
            pltpu.VMEM((KB,), jnp.int32),
            pltpu.VMEM((KB, 128), jnp.float32),
            pltpu.VMEM((128, 128), jnp.float32),
            pltpu.VMEM_SHARED((NP, 128), jnp.float32),
            pltpu.SemaphoreType.DMA,
        ],
    )
    def prop_k(src_hbm, dst_hbm, xs_hbm, out_hbm,
               src_v, idx_v, dst_v, rows_v, zb, acc, sem):
        core = lax.axis_index("c")
        sid = lax.axis_index("s")
        w = core * NS + sid
        rbase = sid * ROWS_PT

        def zrow(i, carry):
            for q in range(8):
                zb[i, pl.ds(q * 16, 16)] = jnp.zeros((16,), jnp.float32)
            return carry

        lax.fori_loop(0, 128, zrow, 0)

        for c in range(C):
            for b in range(ROWS_PT // 128):
                pltpu.sync_copy(zb, acc.at[pl.ds(rbase + b * 128, 128)])
            plsc.subcore_barrier()

            def edge_body(j, carry):
                eb = w * TPC + j * KB
                pltpu.sync_copy(src_hbm.at[pl.ds(eb, KB)], src_v)
                pltpu.sync_copy(dst_hbm.at[pl.ds(eb, KB)], dst_v)
                if C > 1:
                    for i in range(KB // 16):
                        idx_v[pl.ds(i * 16, 16)] = (
                            src_v[pl.ds(i * 16, 16)] * C + c
                        )
                    gidx = idx_v
                else:
                    gidx = src_v
                pltpu.async_copy(xs_hbm.at[gidx], rows_v, sem).wait()
                pltpu.sync_copy(rows_v, acc.at[dst_v], add=True)
                return carry

            lax.fori_loop(0, NB, edge_body, 0)
            plsc.subcore_barrier()
            pltpu.sync_copy(
                acc.at[pl.ds(rbase, ROWS_PT)],
                out_hbm.at[c, core, pl.ds(rbase, ROWS_PT)],
            )

    return prop_k


_prop1 = _build_prop(1)
_prop2 = _build_prop(2)
_prop4 = _build_prop(4)


def _mm_scale(xp, W, dinv2):
    M, Kd = xp.shape
    Dout = W.shape[1]

    def body(x_ref, w_ref, d_ref, o_ref):
        acc = jnp.dot(x_ref[...], w_ref[...], preferred_element_type=jnp.float32)
        o_ref[...] = acc * d_ref[...]

    return pl.pallas_call(
        body,
        grid=(M // BM,),
        in_specs=[
            pl.BlockSpec((BM, Kd), lambda m: (m, 0)),
            pl.BlockSpec((Kd, Dout), lambda m: (0, 0)),
            pl.BlockSpec((BM, 1), lambda m: (m, 0)),
        ],
        out_specs=pl.BlockSpec((BM, Dout), lambda m: (m, 0)),
        out_shape=jax.ShapeDtypeStruct((M, Dout), jnp.float32),
    )(xp, W, dinv2)


def _combine(agg, xs, dinv2, b):
    C = agg.shape[0]
    M, D = xs.shape
    b2 = b.reshape(1, D)

    def body(a_ref, x_ref, d_ref, b_ref, o_ref):
        t = (a_ref[0, 0] + a_ref[0, 1] + x_ref[...]) * d_ref[...] + b_ref[...]
        o_ref[...] = 0.5 * t * (1.0 + lax.erf(t * 0.7071067811865476))

    return pl.pallas_call(
        body,
        grid=(M // BM, C),
        in_specs=[
            pl.BlockSpec((1, 2, BM, 128), lambda m, c: (c, 0, m, 0)),
            pl.BlockSpec((BM, 128), lambda m, c: (m, c)),
            pl.BlockSpec((BM, 1), lambda m, c: (m, 0)),
            pl.BlockSpec((1, 128), lambda m, c: (0, c)),
        ],
        out_specs=pl.BlockSpec((BM, 128), lambda m, c: (m, c)),
        out_shape=jax.ShapeDtypeStruct((M, D), jnp.float32),
    )(agg, xs, dinv2, b2)


def _final_mm(x1, x2, x3, A1, A2, A3, cvec):
    M = x1.shape[0]

    def body(x1r, x2r, x3r, a1r, a2r, a3r, cr, o_ref):
        acc = jnp.dot(x1r[...], a1r[...], preferred_element_type=jnp.float32)
        acc = acc + jnp.dot(x2r[...], a2r[...], preferred_element_type=jnp.float32)
        acc = acc + jnp.dot(x3r[...], a3r[...], preferred_element_type=jnp.float32)
        o_ref[...] = acc + cr[...]

    return pl.pallas_call(
        body,
        grid=(M // BM,),
        in_specs=[
            pl.BlockSpec((BM, 128), lambda m: (m, 0)),
            pl.BlockSpec((BM, 256), lambda m: (m, 0)),
            pl.BlockSpec((BM, 512), lambda m: (m, 0)),
            pl.BlockSpec((128, 128), lambda m: (0, 0)),
            pl.BlockSpec((256, 128), lambda m: (0, 0)),
            pl.BlockSpec((512, 128), lambda m: (0, 0)),
            pl.BlockSpec((1, 128), lambda m: (0, 0)),
        ],
        out_specs=pl.BlockSpec((BM, 128), lambda m: (m, 0)),
        out_shape=jax.ShapeDtypeStruct((M, 128), jnp.float32),
    )(x1, x2, x3, A1, A2, A3, cvec)


def _ctrans(x, W, b):
    y = jnp.einsum('ncl,cok->nolk', x, W)
    n, co, l, k = y.shape
    return y.reshape(n, co, l * k) + b[None, :, None]


def _cproj(x, W, b):
    return jnp.einsum('ncl,oc->nol', x, W[:, :, 0]) + b[None, :, None]


def _decoder_mats(Wd1, bd1, Wd2, bd2, Wd3, bd3, Ws1, bs1, Ws2, bs2, Wf, bf):
    def dec(x1, x2, x3):
        h = _ctrans(x3[:, :, None], Wd1, bd1)
        s1 = _cproj(x2[:, :, None], Ws1, bs1)
        h = h + jnp.tile(s1, (1, 1, h.shape[2]))
        h = _ctrans(h, Wd2, bd2)
        s2 = _cproj(x1[:, :, None], Ws2, bs2)
        h = h + jnp.tile(s2, (1, 1, h.shape[2]))
        h = _ctrans(h, Wd3, bd3)
        return h.reshape(h.shape[0], -1) @ Wf + bf[None, :]

    z = jnp.zeros
    cvec = dec(z((1, 128)), z((1, 256)), z((1, 512)))
    A1 = dec(jnp.eye(128), z((128, 256)), z((128, 512))) - cvec
    A2 = dec(z((256, 128)), jnp.eye(256), z((256, 512))) - cvec
    A3 = dec(z((512, 128)), z((512, 256)), jnp.eye(512)) - cvec
    return A1, A2, A3, cvec


def kernel(x, edge_index, W1, b1, W2, b2, W3, b3, Wd1, bd1, Wd2, bd2, Wd3, bd3,
           Ws1, bs1, Ws2, bs2, Wf, bf):
    src = edge_index[0].astype(jnp.int32)
    dst = edge_index[1].astype(jnp.int32)
    xp = jnp.pad(x, ((0, NP - N), (0, 0)))

    deg = _deg(dst)
    dinv2 = _dinv_tc(deg.reshape(2, NP, 1))
    A1, A2, A3, cvec = _decoder_mats(Wd1, bd1, Wd2, bd2, Wd3, bd3,
                                     Ws1, bs1, Ws2, bs2, Wf, bf)

    xs1 = _mm_scale(xp, W1, dinv2)
    agg1 = _prop1(src, dst, xs1)
    x1 = _combine(agg1, xs1, dinv2, b1)

    xs2 = _mm_scale(x1, W2, dinv2)
    agg2 = _prop2(src, dst, xs2.reshape(NP * 2, 128))
    x2 = _combine(agg2, xs2, dinv2, b2)

    xs3 = _mm_scale(x2, W3, dinv2)
    agg3 = _prop4(src, dst, xs3.reshape(NP * 4, 128))
    x3 = _combine(agg3, xs3, dinv2, b3)

    out = _final_mm(x1, x2, x3, A1, A2, A3, cvec)
    return out[:N]
